# Initial kernel scaffold; baseline (speedup 1.0000x reference)
import jax
import jax.numpy as jnp
from jax.experimental import pallas as pl


def kernel(x, table, bias):
    raise NotImplementedError("write your pallas kernel here")


# speedup vs baseline: 1.1250x; 1.1250x over previous
import functools

import jax
import jax.numpy as jnp
from jax import lax
from jax.experimental import pallas as pl
from jax.experimental.pallas import tpu as pltpu
from jax.experimental.pallas import tpu_sc as plsc

F = 26
B = 16384
V = 100000
NC, NS, L = 2, 16, 16
NW = NC * NS
W = B // NW
GC = 128


def _body(xt_hbm, table_hbm, bias_hbm, out_hbm, xbuf, idxbuf, valbuf, accbuf,
          biasbuf, sem):
    wid = lax.axis_index("s") * NC + lax.axis_index("c")
    base = wid * W

    pltpu.sync_copy(xt_hbm.at[:, pl.ds(base, W)], xbuf)
    pltpu.sync_copy(bias_hbm, biasbuf)

    for v in range(W // L):
        accbuf[pl.ds(v * L, L)] = jnp.zeros((L,), jnp.float32)

    def feature_step(f, carry):
        off = f * V
        for v in range(W // L):
            idxbuf[pl.ds(v * L, L)] = xbuf[f, pl.ds(v * L, L)] + off
        copies = [
            pltpu.async_copy(
                table_hbm.at[idxbuf.at[pl.ds(c * GC, GC)]],
                valbuf.at[pl.ds(c * GC, GC)], sem)
            for c in range(W // GC)
        ]
        for cp in copies:
            cp.wait()
        for v in range(W // L):
            accbuf[pl.ds(v * L, L)] = (accbuf[pl.ds(v * L, L)]
                                       + valbuf[pl.ds(v * L, L)])
        return carry

    lax.fori_loop(0, F, feature_step, 0)

    bias_v = biasbuf[...]
    for v in range(W // L):
        accbuf[pl.ds(v * L, L)] = accbuf[pl.ds(v * L, L)] + bias_v
    pltpu.sync_copy(accbuf, out_hbm.at[pl.ds(base, W)])


@jax.jit
def kernel(x, table, bias):
    xt = x.T.reshape(F, B)
    table_flat = table.reshape(-1)
    bias16 = jnp.broadcast_to(bias, (L,))

    sc_kernel = functools.partial(
        pl.kernel,
        mesh=plsc.VectorSubcoreMesh(core_axis_name="c", subcore_axis_name="s"),
        out_type=jax.ShapeDtypeStruct((B,), jnp.float32),
        scratch_types=[
            pltpu.VMEM((F, W), jnp.int32),
            pltpu.VMEM((W,), jnp.int32),
            pltpu.VMEM((W,), jnp.float32),
            pltpu.VMEM((W,), jnp.float32),
            pltpu.VMEM((L,), jnp.float32),
            pltpu.SemaphoreType.DMA,
        ],
    )(_body)
    out = sc_kernel(xt, table_flat, bias16)
    return out.reshape(B, 1)
